# baseline (speedup 1.0000x reference)
import functools

import jax
import jax.numpy as jnp
from jax import lax
from jax.experimental import pallas as pl
from jax.experimental.pallas import tpu as pltpu
from jax.experimental.pallas import tpu_sc as plsc

N = 10000
K = 8
E = N * K
HN = 64
HE = 64
NAT = 100
_BN = 1.0 / (1.0 + 1e-5) ** 0.5

_NPAD = 10240
_RB = 128
_NT = _NPAD // 128


def _knn_body(posr_ref, post_ref, sqc_ref, sqr_ref, out_ref, d2_ref):
    i = pl.program_id(0)
    dots = jnp.dot(posr_ref[...], post_ref[...],
                   preferred_element_type=jnp.float32)
    colid = jax.lax.broadcasted_iota(jnp.int32, (_RB, _NPAD), 1)
    rowid = i * _RB + jax.lax.broadcasted_iota(jnp.int32, (_RB, _NPAD), 0)
    d2 = (sqr_ref[...] + sqc_ref[...]) - 2.0 * dots
    d2_ref[...] = d2 + jnp.where(colid == rowid, 1e9, 0.0)

    lane = jax.lax.broadcasted_iota(jnp.int32, (_RB, 128), 1)
    big = jnp.float32(3e38)
    imax = jnp.int32(2**31 - 1)
    _U = 8

    def lex_min(av, ai, bv, bi):
        m = (av < bv) | ((av == bv) & (ai < bi))
        return jnp.where(m, av, bv), jnp.where(m, ai, bi)

    pv = jnp.full((_RB, 1), -big, jnp.float32)
    pg = jnp.full((_RB, 1), -1, jnp.int32)
    for k in range(K // 2):
        def fold(t, carry):
            m1, i1, m2, i2 = carry
            for u in range(_U):
                v = d2_ref[:, pl.ds((t * _U + u) * 128, 128)]
                gidx = (t * _U + u) * 128 + lane
                adm = (v > pv) | ((v == pv) & (gidx > pg))
                v = jnp.where(adm, v, big)
                lt1 = v < m1
                lt2 = v < m2
                m2 = jnp.where(lt1, m1, jnp.where(lt2, v, m2))
                i2 = jnp.where(lt1, i1, jnp.where(lt2, gidx, i2))
                m1 = jnp.where(lt1, v, m1)
                i1 = jnp.where(lt1, gidx, i1)
            return m1, i1, m2, i2
        f0 = jnp.full((_RB, 128), big, jnp.float32)
        g0 = jnp.full((_RB, 128), imax, jnp.int32)
        m1, i1, m2, i2 = jax.lax.fori_loop(0, _NT // _U, fold,
                                           (f0, g0, f0 + 0.0, g0 + 0))
        v1 = jnp.min(m1, axis=1, keepdims=True)
        s1 = jnp.min(jnp.where(m1 == v1, i1, imax), axis=1, keepdims=True)
        hit = (m1 == v1) & (i1 == s1)
        r1 = jnp.where(hit, m2, m1)
        r1i = jnp.where(hit, i2, i1)
        v2 = jnp.min(r1, axis=1, keepdims=True)
        s2 = jnp.min(jnp.where(r1 == v2, r1i, imax), axis=1, keepdims=True)
        out_ref[:, 2 * k:2 * k + 1] = s1
        out_ref[:, 2 * k + 1:2 * k + 2] = s2
        pv, pg = v2, s2


def _knn(pos, sq):
    posr = jnp.zeros((_NPAD, 8), jnp.float32).at[:N, :3].set(pos)
    post = jnp.zeros((8, _NPAD), jnp.float32).at[:3, :N].set(pos.T)
    sqp = jnp.full((_NPAD,), 4e9, jnp.float32).at[:N].set(sq)
    nbr = pl.pallas_call(
        _knn_body,
        grid=(_NPAD // _RB,),
        in_specs=[
            pl.BlockSpec((_RB, 8), lambda i: (i, 0)),
            pl.BlockSpec((8, _NPAD), lambda i: (0, 0)),
            pl.BlockSpec((1, _NPAD), lambda i: (0, 0)),
            pl.BlockSpec((_RB, 1), lambda i: (i, 0)),
        ],
        out_specs=pl.BlockSpec((_RB, K), lambda i: (i, 0)),
        out_shape=jax.ShapeDtypeStruct((_NPAD, K), jnp.int32),
        scratch_shapes=[pltpu.VMEM((_RB, _NPAD), jnp.float32)],
    )(posr, post, sqp.reshape(1, _NPAD), sqp.reshape(_NPAD, 1))
    return nbr[:N]


_SC_NC, _SC_NS = 2, 16
_SC_NW = _SC_NC * _SC_NS
_EPAD = 81920
_BPW = _EPAD // _SC_NW


def _sc_gather(table, idx_w, d, ch):
    nchunk = _BPW // ch
    mesh = plsc.VectorSubcoreMesh(core_axis_name="c", subcore_axis_name="s")

    @functools.partial(
        pl.kernel, mesh=mesh,
        out_type=jax.ShapeDtypeStruct((_EPAD, d), jnp.float32),
        scratch_types=[
            pltpu.VMEM((nchunk, ch), jnp.int32),
            pltpu.VMEM((2, ch, d), jnp.float32),
            pltpu.SemaphoreType.DMA((2,)),
            pltpu.SemaphoreType.DMA((2,)),
        ],
    )
    def k(table_hbm, idx_hbm, out_hbm, idx_v, rows_v, gsem, osem):
        wid = lax.axis_index("s") * _SC_NC + lax.axis_index("c")
        base = wid * _BPW
        pltpu.sync_copy(idx_hbm.at[wid], idx_v)

        def start_g(cc, b):
            pltpu.async_copy(table_hbm.at[idx_v.at[cc]], rows_v.at[b],
                             gsem.at[b])

        def wait_g(cc, b):
            pltpu.make_async_copy(table_hbm.at[idx_v.at[cc]], rows_v.at[b],
                                  gsem.at[b]).wait()

        def start_o(cc, b):
            pltpu.async_copy(rows_v.at[b],
                             out_hbm.at[pl.ds(base + cc * ch, ch)], osem.at[b])

        def wait_o(cc, b):
            pltpu.make_async_copy(rows_v.at[b],
                                  out_hbm.at[pl.ds(base + cc * ch, ch)],
                                  osem.at[b]).wait()

        for b in range(2):
            start_g(b, b)

        @pl.loop(0, nchunk - 2, step=2)
        def _chunk(c):
            for b in range(2):
                cc = c + b
                wait_g(cc, b)
                start_o(cc, b)
                wait_o(cc, b)
                start_g(cc + 2, b)

        for b in range(2):
            cc = nchunk - 2 + b
            wait_g(cc, b)
            start_o(cc, b)
        for b in range(2):
            wait_o(nchunk - 2 + b, b)

    return k(table, idx_w)


def _gather_rows(table, idx_w, ch):
    return _sc_gather(table, idx_w, table.shape[1], ch)


def _ssp(x):
    return jax.nn.softplus(x) - jnp.log(2.0)


_BE = 2048


def _c3_body(g_ref, d_ref, m_ref, o_ref):
    d = d_ref[...]
    acc = jnp.zeros((_BE, HE), jnp.float32)
    for q in range(K):
        pre = (g_ref[:, q * 128:(q + 1) * 128] + d) * _BN
        f = pre[:, :HE]
        c = pre[:, HE:]
        acc += jax.nn.sigmoid(f) * jnp.tanh(c) * m_ref[:, q:q + 1]
    o_ref[...] = acc * _BN


def _c3_sum(G, D, mask):
    return pl.pallas_call(
        _c3_body,
        grid=(_EPAD // _BE,),
        in_specs=[
            pl.BlockSpec((_BE, K * 128), lambda i: (i, 0)),
            pl.BlockSpec((_BE, 128), lambda i: (i, 0)),
            pl.BlockSpec((_BE, K), lambda i: (i, 0)),
        ],
        out_specs=pl.BlockSpec((_BE, HE), lambda i: (i, 0)),
        out_shape=jax.ShapeDtypeStruct((_EPAD, HE), jnp.float32),
    )(G, D, mask)


def kernel(z, pos, params):
    sq = jnp.sum(pos * pos, axis=1)
    nbr = _knn(pos, sq)
    row = nbr.reshape(-1)
    col = jnp.repeat(jnp.arange(N), K)

    pos_col = jnp.repeat(pos, K, axis=0)

    W, b = params['emb0']
    h = _ssp(jax.nn.one_hot(z - 1, NAT, dtype=jnp.float32) @ W + b)
    W, b = params['emb1']
    h = _ssp(h @ W + b)
    W, b = params['emb2']
    node = h @ W + b

    row_pad = jnp.concatenate(
        [row, jnp.arange(_EPAD - E, dtype=jnp.int32) * 5 % N])
    row_w32 = row_pad.reshape(_SC_NW, _BPW // 32, 32)
    row_w128 = row_pad.reshape(_SC_NW, _BPW // 128, 128)
    T0 = jnp.zeros((N, 128), jnp.float32)
    T0 = T0.at[:, :3].set(pos).at[:, 3:3 + K].set(nbr.astype(jnp.float32))
    G0 = _gather_rows(T0, row_w128, 128)[:E]
    rel = pos_col - G0[:, :3]
    dist = jnp.sqrt(jnp.sum(rel * rel, axis=-1))
    unit = rel / dist[:, None]

    offset = jnp.linspace(0.0, 5.0, HE)
    coeff = -0.5 / (offset[1] - offset[0]) ** 2
    edge = jnp.exp(coeff * (dist[:, None] - offset[None, :]) ** 2)

    colf = col.astype(jnp.float32)
    mask = (colf[:, None] != G0[:, 3:3 + K]).astype(jnp.float32)
    mask_pad = jnp.concatenate(
        [mask, jnp.zeros((_EPAD - E, K), jnp.float32)])

    for lp in params['layers']:
        W, b = lp['nu']
        pre = (jnp.repeat(node @ W[:HN], K, axis=0) + edge @ W[HN:] + b) * _BN
        gated = jax.nn.sigmoid(pre[:, :HN]) * jnp.tanh(pre[:, HN:])
        agg = gated.reshape(N, K, HN).sum(axis=1)
        node = jnp.tanh(node + agg * _BN)

        W, b = lp['c3']
        Wi, Wj, Wk = W[:HN], W[HN:2 * HN], W[2 * HN:3 * HN]
        Wji, Wkj = W[3 * HN:3 * HN + HE], W[3 * HN + HE:]

        T = jnp.concatenate(
            [node, node @ Wk, jnp.zeros((N, 64), jnp.float32)], axis=1)
        TG = _gather_rows(T, row_w128, 128)[:E]

        W2, b2 = lp['c2']
        prod = jnp.repeat(node, K, axis=0) * TG[:, :HN]
        c2 = (prod @ W2 + b2) * _BN
        c2e = jax.nn.sigmoid(c2[:, :HE]) * jnp.tanh(c2[:, HE:]) * _BN

        D = jnp.repeat(node @ Wi, K, axis=0) + edge @ Wji + b
        Dp = jnp.concatenate([D, jnp.zeros((_EPAD - E, 128), jnp.float32)])
        S = jnp.repeat(node @ Wj, K, axis=0) + TG[:, HN:HN + 128] + edge @ Wkj
        G = _gather_rows(S.reshape(N, K * 128), row_w32, 32)
        c3e = _c3_sum(G, Dp, mask_pad)[:E]

        edge = jnp.tanh(edge + c2e + c3e)

    W, b = params['fp0']
    h = _ssp(edge @ W + b)
    W, b = params['fp1']
    h = _ssp(h @ W + b)
    W, b = params['fp2']
    s = h @ W + b
    force = s * unit
    return force.reshape(N, K, 3).sum(axis=1)


# speedup vs baseline: 1.0562x; 1.0562x over previous
import functools

import jax
import jax.numpy as jnp
from jax import lax
from jax.experimental import pallas as pl
from jax.experimental.pallas import tpu as pltpu
from jax.experimental.pallas import tpu_sc as plsc

N = 10000
K = 8
E = N * K
HN = 64
HE = 64
NAT = 100
_BN = 1.0 / (1.0 + 1e-5) ** 0.5

_NPAD = 10240
_RB = 64
_NT = _NPAD // 128


def _knn_body(posr_ref, post_ref, sqc_ref, sqr_ref, out_ref, d2_ref):
    i = pl.program_id(0)
    dots = jnp.dot(posr_ref[...], post_ref[...],
                   preferred_element_type=jnp.float32)
    colid = jax.lax.broadcasted_iota(jnp.int32, (_RB, _NPAD), 1)
    rowid = i * _RB + jax.lax.broadcasted_iota(jnp.int32, (_RB, _NPAD), 0)
    d2 = (sqr_ref[...] + sqc_ref[...]) - 2.0 * dots
    d2_ref[...] = d2 + jnp.where(colid == rowid, 1e9, 0.0)

    lane = jax.lax.broadcasted_iota(jnp.int32, (_RB, 128), 1)
    big = jnp.float32(3e38)
    imax = jnp.int32(2**31 - 1)
    _U = 8

    def lex_min(av, ai, bv, bi):
        m = (av < bv) | ((av == bv) & (ai < bi))
        return jnp.where(m, av, bv), jnp.where(m, ai, bi)

    pv = jnp.full((_RB, 1), -big, jnp.float32)
    pg = jnp.full((_RB, 1), -1, jnp.int32)
    for k in range(K // 2):
        def fold(t, carry):
            m1, i1, m2, i2 = carry
            for u in range(_U):
                v = d2_ref[:, pl.ds((t * _U + u) * 128, 128)]
                gidx = (t * _U + u) * 128 + lane
                if k > 0:
                    adm = (v > pv) | ((v == pv) & (gidx > pg))
                    v = jnp.where(adm, v, big)
                lt1 = v < m1
                lt2 = v < m2
                m2 = jnp.where(lt1, m1, jnp.where(lt2, v, m2))
                i2 = jnp.where(lt1, i1, jnp.where(lt2, gidx, i2))
                m1 = jnp.where(lt1, v, m1)
                i1 = jnp.where(lt1, gidx, i1)
            return m1, i1, m2, i2
        f0 = jnp.full((_RB, 128), big, jnp.float32)
        g0 = jnp.full((_RB, 128), imax, jnp.int32)
        m1, i1, m2, i2 = jax.lax.fori_loop(0, _NT // _U, fold,
                                           (f0, g0, f0 + 0.0, g0 + 0))
        v1 = jnp.min(m1, axis=1, keepdims=True)
        s1 = jnp.min(jnp.where(m1 == v1, i1, imax), axis=1, keepdims=True)
        hit = (m1 == v1) & (i1 == s1)
        r1 = jnp.where(hit, m2, m1)
        r1i = jnp.where(hit, i2, i1)
        v2 = jnp.min(r1, axis=1, keepdims=True)
        s2 = jnp.min(jnp.where(r1 == v2, r1i, imax), axis=1, keepdims=True)
        out_ref[:, 2 * k:2 * k + 1] = s1
        out_ref[:, 2 * k + 1:2 * k + 2] = s2
        pv, pg = v2, s2


def _knn(pos, sq):
    posr = jnp.zeros((_NPAD, 8), jnp.float32).at[:N, :3].set(pos)
    post = jnp.zeros((8, _NPAD), jnp.float32).at[:3, :N].set(pos.T)
    sqp = jnp.full((_NPAD,), 4e9, jnp.float32).at[:N].set(sq)
    nbr = pl.pallas_call(
        _knn_body,
        grid=(_NPAD // _RB,),
        in_specs=[
            pl.BlockSpec((_RB, 8), lambda i: (i, 0)),
            pl.BlockSpec((8, _NPAD), lambda i: (0, 0)),
            pl.BlockSpec((1, _NPAD), lambda i: (0, 0)),
            pl.BlockSpec((_RB, 1), lambda i: (i, 0)),
        ],
        out_specs=pl.BlockSpec((_RB, K), lambda i: (i, 0)),
        out_shape=jax.ShapeDtypeStruct((_NPAD, K), jnp.int32),
        scratch_shapes=[pltpu.VMEM((_RB, _NPAD), jnp.float32)],
    )(posr, post, sqp.reshape(1, _NPAD), sqp.reshape(_NPAD, 1))
    return nbr[:N]


_SC_NC, _SC_NS = 2, 16
_SC_NW = _SC_NC * _SC_NS
_EPAD = 81920
_BPW = _EPAD // _SC_NW


def _sc_gather(table, idx_w, d, ch):
    nchunk = _BPW // ch
    mesh = plsc.VectorSubcoreMesh(core_axis_name="c", subcore_axis_name="s")

    @functools.partial(
        pl.kernel, mesh=mesh,
        out_type=jax.ShapeDtypeStruct((_EPAD, d), jnp.float32),
        scratch_types=[
            pltpu.VMEM((nchunk, ch), jnp.int32),
            pltpu.VMEM((2, ch, d), jnp.float32),
            pltpu.SemaphoreType.DMA((2,)),
            pltpu.SemaphoreType.DMA((2,)),
        ],
    )
    def k(table_hbm, idx_hbm, out_hbm, idx_v, rows_v, gsem, osem):
        wid = lax.axis_index("s") * _SC_NC + lax.axis_index("c")
        base = wid * _BPW
        pltpu.sync_copy(idx_hbm.at[wid], idx_v)

        def start_g(cc, b):
            pltpu.async_copy(table_hbm.at[idx_v.at[cc]], rows_v.at[b],
                             gsem.at[b])

        def wait_g(cc, b):
            pltpu.make_async_copy(table_hbm.at[idx_v.at[cc]], rows_v.at[b],
                                  gsem.at[b]).wait()

        def start_o(cc, b):
            pltpu.async_copy(rows_v.at[b],
                             out_hbm.at[pl.ds(base + cc * ch, ch)], osem.at[b])

        def wait_o(cc, b):
            pltpu.make_async_copy(rows_v.at[b],
                                  out_hbm.at[pl.ds(base + cc * ch, ch)],
                                  osem.at[b]).wait()

        for b in range(2):
            start_g(b, b)

        @pl.loop(0, nchunk - 2, step=2)
        def _chunk(c):
            for b in range(2):
                cc = c + b
                wait_g(cc, b)
                start_o(cc, b)
                wait_o(cc, b)
                start_g(cc + 2, b)

        for b in range(2):
            cc = nchunk - 2 + b
            wait_g(cc, b)
            start_o(cc, b)
        for b in range(2):
            wait_o(nchunk - 2 + b, b)

    return k(table, idx_w)


def _gather_rows(table, idx_w, ch):
    return _sc_gather(table, idx_w, table.shape[1], ch)


def _ssp(x):
    return jax.nn.softplus(x) - jnp.log(2.0)


_BE = 2048


def _c3_body(g_ref, d_ref, m_ref, o_ref):
    d = d_ref[...]
    acc = jnp.zeros((_BE, HE), jnp.float32)
    for q in range(K):
        pre = (g_ref[:, q * 128:(q + 1) * 128] + d) * _BN
        f = pre[:, :HE]
        c = pre[:, HE:]
        acc += jax.nn.sigmoid(f) * jnp.tanh(c) * m_ref[:, q:q + 1]
    o_ref[...] = acc * _BN


def _c3_sum(G, D, mask):
    return pl.pallas_call(
        _c3_body,
        grid=(_EPAD // _BE,),
        in_specs=[
            pl.BlockSpec((_BE, K * 128), lambda i: (i, 0)),
            pl.BlockSpec((_BE, 128), lambda i: (i, 0)),
            pl.BlockSpec((_BE, K), lambda i: (i, 0)),
        ],
        out_specs=pl.BlockSpec((_BE, HE), lambda i: (i, 0)),
        out_shape=jax.ShapeDtypeStruct((_EPAD, HE), jnp.float32),
    )(G, D, mask)


def kernel(z, pos, params):
    sq = jnp.sum(pos * pos, axis=1)
    nbr = _knn(pos, sq)
    row = nbr.reshape(-1)
    col = jnp.repeat(jnp.arange(N), K)

    pos_col = jnp.repeat(pos, K, axis=0)

    W, b = params['emb0']
    h = _ssp(jax.nn.one_hot(z - 1, NAT, dtype=jnp.float32) @ W + b)
    W, b = params['emb1']
    h = _ssp(h @ W + b)
    W, b = params['emb2']
    node = h @ W + b

    row_pad = jnp.concatenate(
        [row, jnp.arange(_EPAD - E, dtype=jnp.int32) * 5 % N])
    row_w32 = row_pad.reshape(_SC_NW, _BPW // 32, 32)
    row_w128 = row_pad.reshape(_SC_NW, _BPW // 128, 128)
    T0 = jnp.zeros((N, 128), jnp.float32)
    T0 = T0.at[:, :3].set(pos).at[:, 3:3 + K].set(nbr.astype(jnp.float32))
    G0 = _gather_rows(T0, row_w128, 128)[:E]
    rel = pos_col - G0[:, :3]
    dist = jnp.sqrt(jnp.sum(rel * rel, axis=-1))
    unit = rel / dist[:, None]

    offset = jnp.linspace(0.0, 5.0, HE)
    coeff = -0.5 / (offset[1] - offset[0]) ** 2
    edge = jnp.exp(coeff * (dist[:, None] - offset[None, :]) ** 2)

    colf = col.astype(jnp.float32)
    mask = (colf[:, None] != G0[:, 3:3 + K]).astype(jnp.float32)
    mask_pad = jnp.concatenate(
        [mask, jnp.zeros((_EPAD - E, K), jnp.float32)])

    for lp in params['layers']:
        W, b = lp['nu']
        pre = (jnp.repeat(node @ W[:HN], K, axis=0) + edge @ W[HN:] + b) * _BN
        gated = jax.nn.sigmoid(pre[:, :HN]) * jnp.tanh(pre[:, HN:])
        agg = gated.reshape(N, K, HN).sum(axis=1)
        node = jnp.tanh(node + agg * _BN)

        W, b = lp['c3']
        Wi, Wj, Wk = W[:HN], W[HN:2 * HN], W[2 * HN:3 * HN]
        Wji, Wkj = W[3 * HN:3 * HN + HE], W[3 * HN + HE:]

        T = jnp.concatenate(
            [node, node @ Wk, jnp.zeros((N, 64), jnp.float32)], axis=1)
        TG = _gather_rows(T, row_w128, 128)[:E]

        W2, b2 = lp['c2']
        prod = jnp.repeat(node, K, axis=0) * TG[:, :HN]
        c2 = (prod @ W2 + b2) * _BN
        c2e = jax.nn.sigmoid(c2[:, :HE]) * jnp.tanh(c2[:, HE:]) * _BN

        D = jnp.repeat(node @ Wi, K, axis=0) + edge @ Wji + b
        Dp = jnp.concatenate([D, jnp.zeros((_EPAD - E, 128), jnp.float32)])
        S = jnp.repeat(node @ Wj, K, axis=0) + TG[:, HN:HN + 128] + edge @ Wkj
        G = _gather_rows(S.reshape(N, K * 128), row_w32, 32)
        c3e = _c3_sum(G, Dp, mask_pad)[:E]

        edge = jnp.tanh(edge + c2e + c3e)

    W, b = params['fp0']
    h = _ssp(edge @ W + b)
    W, b = params['fp1']
    h = _ssp(h @ W + b)
    W, b = params['fp2']
    s = h @ W + b
    force = s * unit
    return force.reshape(N, K, 3).sum(axis=1)


# speedup vs baseline: 1.0568x; 1.0005x over previous
import functools

import jax
import jax.numpy as jnp
from jax import lax
from jax.experimental import pallas as pl
from jax.experimental.pallas import tpu as pltpu
from jax.experimental.pallas import tpu_sc as plsc

N = 10000
K = 8
E = N * K
HN = 64
HE = 64
NAT = 100
_BN = 1.0 / (1.0 + 1e-5) ** 0.5

_NPAD = 10240
_RB = 64
_NT = _NPAD // 128


def _knn_body(posr_ref, post_ref, sqc_ref, sqr_ref, out_ref, d2_ref):
    i = pl.program_id(0)
    dots = jnp.dot(posr_ref[...], post_ref[...],
                   preferred_element_type=jnp.float32)
    colid = jax.lax.broadcasted_iota(jnp.int32, (_RB, _NPAD), 1)
    rowid = i * _RB + jax.lax.broadcasted_iota(jnp.int32, (_RB, _NPAD), 0)
    d2 = (sqr_ref[...] + sqc_ref[...]) - 2.0 * dots
    d2_ref[...] = d2 + jnp.where(colid == rowid, 1e9, 0.0)

    lane = jax.lax.broadcasted_iota(jnp.int32, (_RB, 128), 1)
    big = jnp.float32(3e38)
    imax = jnp.int32(2**31 - 1)
    _U = 8

    pv = jnp.full((_RB, 1), -big, jnp.float32)
    pg = jnp.full((_RB, 1), -1, jnp.int32)
    for k in range(K // 2):
        def fold(t, carry):
            m1, i1, m2, i2 = carry
            for u in range(_U):
                v = d2_ref[:, pl.ds((t * _U + u) * 128, 128)]
                gidx = (t * _U + u) * 128 + lane
                if k > 0:
                    adm = (v > pv) | ((v == pv) & (gidx > pg))
                    v = jnp.where(adm, v, big)
                lt1 = v < m1
                lt2 = v < m2
                m2 = jnp.where(lt1, m1, jnp.where(lt2, v, m2))
                i2 = jnp.where(lt1, i1, jnp.where(lt2, gidx, i2))
                m1 = jnp.where(lt1, v, m1)
                i1 = jnp.where(lt1, gidx, i1)
            return m1, i1, m2, i2
        f0 = jnp.full((_RB, 128), big, jnp.float32)
        g0 = jnp.full((_RB, 128), imax, jnp.int32)
        m1, i1, m2, i2 = jax.lax.fori_loop(0, _NT // _U, fold,
                                           (f0, g0, f0 + 0.0, g0 + 0))
        v1 = jnp.min(m1, axis=1, keepdims=True)
        s1 = jnp.min(jnp.where(m1 == v1, i1, imax), axis=1, keepdims=True)
        hit = (m1 == v1) & (i1 == s1)
        r1 = jnp.where(hit, m2, m1)
        r1i = jnp.where(hit, i2, i1)
        v2 = jnp.min(r1, axis=1, keepdims=True)
        s2 = jnp.min(jnp.where(r1 == v2, r1i, imax), axis=1, keepdims=True)
        out_ref[:, 2 * k:2 * k + 1] = s1
        out_ref[:, 2 * k + 1:2 * k + 2] = s2
        pv, pg = v2, s2


def _knn(pos, sq):
    posr = jnp.zeros((_NPAD, 8), jnp.float32).at[:N, :3].set(pos)
    post = jnp.zeros((8, _NPAD), jnp.float32).at[:3, :N].set(pos.T)
    sqp = jnp.full((_NPAD,), 4e9, jnp.float32).at[:N].set(sq)
    nbr = pl.pallas_call(
        _knn_body,
        grid=(_NPAD // _RB,),
        in_specs=[
            pl.BlockSpec((_RB, 8), lambda i: (i, 0)),
            pl.BlockSpec((8, _NPAD), lambda i: (0, 0)),
            pl.BlockSpec((1, _NPAD), lambda i: (0, 0)),
            pl.BlockSpec((_RB, 1), lambda i: (i, 0)),
        ],
        out_specs=pl.BlockSpec((_RB, K), lambda i: (i, 0)),
        out_shape=jax.ShapeDtypeStruct((_NPAD, K), jnp.int32),
        scratch_shapes=[pltpu.VMEM((_RB, _NPAD), jnp.float32)],
    )(posr, post, sqp.reshape(1, _NPAD), sqp.reshape(_NPAD, 1))
    return nbr[:N]


_SC_NC, _SC_NS = 2, 16
_SC_NW = _SC_NC * _SC_NS
_EPAD = 81920
_BPW = _EPAD // _SC_NW


def _sc_gather(table, idx_w, d, ch):
    nchunk = _BPW // ch
    mesh = plsc.VectorSubcoreMesh(core_axis_name="c", subcore_axis_name="s")

    @functools.partial(
        pl.kernel, mesh=mesh,
        out_type=jax.ShapeDtypeStruct((_EPAD, d), jnp.float32),
        scratch_types=[
            pltpu.VMEM((nchunk, ch), jnp.int32),
            pltpu.VMEM((2, ch, d), jnp.float32),
            pltpu.SemaphoreType.DMA((2,)),
            pltpu.SemaphoreType.DMA((2,)),
        ],
    )
    def k(table_hbm, idx_hbm, out_hbm, idx_v, rows_v, gsem, osem):
        wid = lax.axis_index("s") * _SC_NC + lax.axis_index("c")
        base = wid * _BPW
        pltpu.sync_copy(idx_hbm.at[wid], idx_v)

        def start_g(cc, b):
            pltpu.async_copy(table_hbm.at[idx_v.at[cc]], rows_v.at[b],
                             gsem.at[b])

        def wait_g(cc, b):
            pltpu.make_async_copy(table_hbm.at[idx_v.at[cc]], rows_v.at[b],
                                  gsem.at[b]).wait()

        def start_o(cc, b):
            pltpu.async_copy(rows_v.at[b],
                             out_hbm.at[pl.ds(base + cc * ch, ch)], osem.at[b])

        def wait_o(cc, b):
            pltpu.make_async_copy(rows_v.at[b],
                                  out_hbm.at[pl.ds(base + cc * ch, ch)],
                                  osem.at[b]).wait()

        for b in range(2):
            start_g(b, b)

        @pl.loop(0, nchunk - 2, step=2)
        def _chunk(c):
            for b in range(2):
                cc = c + b
                wait_g(cc, b)
                start_o(cc, b)
                wait_o(cc, b)
                start_g(cc + 2, b)

        for b in range(2):
            cc = nchunk - 2 + b
            wait_g(cc, b)
            start_o(cc, b)
        for b in range(2):
            wait_o(nchunk - 2 + b, b)

    return k(table, idx_w)


def _gather_rows(table, idx_w, ch):
    return _sc_gather(table, idx_w, table.shape[1], ch)


def _ssp(x):
    return jax.nn.softplus(x) - jnp.log(2.0)


_BE = 2048


def _c3_body(g_ref, d_ref, m_ref, o_ref):
    d = d_ref[...]
    acc = jnp.zeros((_BE, HE), jnp.float32)
    for q in range(K):
        pre = (g_ref[:, q * 128:(q + 1) * 128] + d) * _BN
        f = pre[:, :HE]
        c = pre[:, HE:]
        acc += jax.nn.sigmoid(f) * jnp.tanh(c) * m_ref[:, q:q + 1]
    o_ref[...] = acc * _BN


def _c3_sum(G, D, mask):
    return pl.pallas_call(
        _c3_body,
        grid=(_EPAD // _BE,),
        in_specs=[
            pl.BlockSpec((_BE, K * 128), lambda i: (i, 0)),
            pl.BlockSpec((_BE, 128), lambda i: (i, 0)),
            pl.BlockSpec((_BE, K), lambda i: (i, 0)),
        ],
        out_specs=pl.BlockSpec((_BE, HE), lambda i: (i, 0)),
        out_shape=jax.ShapeDtypeStruct((_EPAD, HE), jnp.float32),
    )(G, D, mask)


def kernel(z, pos, params):
    sq = jnp.sum(pos * pos, axis=1)
    nbr = _knn(pos, sq)
    row = nbr.reshape(-1)
    col = jnp.repeat(jnp.arange(N), K)

    pos_col = jnp.repeat(pos, K, axis=0)

    W, b = params['emb0']
    h = _ssp(jax.nn.one_hot(z - 1, NAT, dtype=jnp.float32) @ W + b)
    W, b = params['emb1']
    h = _ssp(h @ W + b)
    W, b = params['emb2']
    node = h @ W + b

    row_pad = jnp.concatenate(
        [row, jnp.arange(_EPAD - E, dtype=jnp.int32) * 5 % N])
    row_w32 = row_pad.reshape(_SC_NW, _BPW // 32, 32)
    row_w128 = row_pad.reshape(_SC_NW, _BPW // 128, 128)
    T0 = jnp.zeros((N, 128), jnp.float32)
    T0 = T0.at[:, :3].set(pos).at[:, 3:3 + K].set(nbr.astype(jnp.float32))
    G0 = _gather_rows(T0, row_w128, 128)[:E]
    rel = pos_col - G0[:, :3]
    dist = jnp.sqrt(jnp.sum(rel * rel, axis=-1))
    unit = rel / dist[:, None]

    offset = jnp.linspace(0.0, 5.0, HE)
    coeff = -0.5 / (offset[1] - offset[0]) ** 2
    edge = jnp.exp(coeff * (dist[:, None] - offset[None, :]) ** 2)

    colf = col.astype(jnp.float32)
    mask = (colf[:, None] != G0[:, 3:3 + K]).astype(jnp.float32)
    mask_pad = jnp.concatenate(
        [mask, jnp.zeros((_EPAD - E, K), jnp.float32)])

    for lp in params['layers']:
        W, b = lp['nu']
        pre = (jnp.repeat(node @ W[:HN], K, axis=0) + edge @ W[HN:] + b) * _BN
        gated = jax.nn.sigmoid(pre[:, :HN]) * jnp.tanh(pre[:, HN:])
        agg = gated.reshape(N, K, HN).sum(axis=1)
        node = jnp.tanh(node + agg * _BN)

        W, b = lp['c3']
        Wi, Wj, Wk = W[:HN], W[HN:2 * HN], W[2 * HN:3 * HN]
        Wji, Wkj = W[3 * HN:3 * HN + HE], W[3 * HN + HE:]

        T = jnp.concatenate(
            [node, node @ Wk, jnp.zeros((N, 64), jnp.float32)], axis=1)
        TG = _gather_rows(T, row_w128, 128)[:E]

        W2, b2 = lp['c2']
        prod = jnp.repeat(node, K, axis=0) * TG[:, :HN]
        c2 = (prod @ W2 + b2) * _BN
        c2e = jax.nn.sigmoid(c2[:, :HE]) * jnp.tanh(c2[:, HE:]) * _BN

        D = jnp.repeat(node @ Wi, K, axis=0) + edge @ Wji + b
        Dp = jnp.concatenate([D, jnp.zeros((_EPAD - E, 128), jnp.float32)])
        S = jnp.repeat(node @ Wj, K, axis=0) + TG[:, HN:HN + 128] + edge @ Wkj
        G = _gather_rows(S.reshape(N, K * 128), row_w32, 32)
        c3e = _c3_sum(G, Dp, mask_pad)[:E]

        edge = jnp.tanh(edge + c2e + c3e)

    W, b = params['fp0']
    h = _ssp(edge @ W + b)
    W, b = params['fp1']
    h = _ssp(h @ W + b)
    W, b = params['fp2']
    s = h @ W + b
    force = s * unit
    return force.reshape(N, K, 3).sum(axis=1)
